# baseline (speedup 1.0000x reference)
import jax
import jax.numpy as jnp
from jax import lax
from jax.experimental import pallas as pl
from jax.experimental.pallas import tpu as pltpu
from jax.experimental.pallas import tpu_sc as plsc

N_MEM = 100000
H = 128
TOPK = 32
BLK_ROWS = 8192
N_PAD = 100352
GRID = (N_MEM + BLK_ROWS - 1) // BLK_ROWS
NEG = -1e30


def _score_body(x_ref, mem_ref, out_ref, t_ref):
    pid = pl.program_id(0)

    @pl.when(pid == 0)
    def _():
        t_ref[...] = jnp.mean(x_ref[...], axis=0, keepdims=True)

    t = t_ref[...]
    m = mem_ref[...]
    num = jnp.sum(m * t, axis=1)
    ss = jnp.sum(m * m, axis=1)
    s = num * jax.lax.rsqrt(jnp.maximum(ss, jnp.float32(1e-16)))
    flat = pid * BLK_ROWS + jax.lax.iota(jnp.int32, BLK_ROWS)
    out_ref[...] = jnp.where(flat < N_MEM, s, NEG)


def _scores(x, memory):
    return pl.pallas_call(
        _score_body,
        grid=(GRID,),
        in_specs=[
            pl.BlockSpec((1024, H), lambda i: (0, 0)),
            pl.BlockSpec((BLK_ROWS, H), lambda i: (i, 0)),
        ],
        out_specs=pl.BlockSpec((BLK_ROWS,), lambda i: (i,)),
        out_shape=jax.ShapeDtypeStruct((N_PAD,), jnp.float32),
        scratch_shapes=[pltpu.VMEM((1, H), jnp.float32)],
    )(x, memory)


NT = 16
PER = N_PAD // NT
NV = PER // 16
CAP = 64
MCAP = 1024
BIG = 2**31 - 1


def _sc_body(scores_hbm, mem_hbm, out_hbm,
             chunk_v, keys_v, hist1, hist2, cand_val, cand_idx,
             shared_val, shared_idx, merge_val, merge_idx, merge_keys,
             mcand_val, mcand_idx, idx32, rows_v, sem):
    wid = lax.axis_index("s")
    lane = lax.iota(jnp.int32, 16)
    ones = lane * 0 + 1

    def keyify(v):
        u = lax.bitcast_convert_type(v, jnp.int32)
        return u ^ (lax.shift_right_arithmetic(u, 31) & jnp.int32(0x7FFFFFFF))

    def zero_hists():
        z = lane * 0
        for c in range(256):
            hist1[pl.ds(c * 16, 16)] = z
            hist2[pl.ds(c * 16, 16)] = z

    def suffix_scan(hist, need):
        def chunk_step(c2, carry):
            s_hi, found, bstar, n_above = carry
            c = 15 - c2
            acc = lane * 0
            for l in range(16):
                acc = acc + hist[pl.ds(l * 256 + c * 16, 16)]
            cs = plsc.cumsum(lax.rev(acc, dimensions=(0,)))
            tot = lax.reduce_max(cs, (0,))
            mask = (s_hi + cs) >= need
            anyv = lax.reduce_max(mask.astype(jnp.int32), (0,))
            j = lax.reduce_max(plsc.all_reduce_ffs(mask), (0,))
            below = jnp.where(lane < j, cs, lane * 0)
            n_ab = s_hi + lax.reduce_max(below, (0,))
            upd = (1 - found) * anyv
            bstar = jnp.where(upd == 1, c * 16 + 15 - j, bstar)
            n_above = jnp.where(upd == 1, n_ab, n_above)
            found = jnp.maximum(found, anyv)
            return (s_hi + tot, found, bstar, n_above)

        init = (jnp.int32(0), jnp.int32(0), jnp.int32(0), jnp.int32(0))
        _, _, bstar, n_above = lax.fori_loop(0, 16, chunk_step, init)
        return bstar, n_above

    def select_threshold(read_val, keys_ref, n, need):
        def step1(i, _):
            k = keyify(read_val(i))
            keys_ref[pl.ds(i * 16, 16)] = k
            b = lax.shift_right_arithmetic(k, 24) + 128
            plsc.addupdate_scatter(hist1, [lane * 256 + b], ones)
            return 0
        lax.fori_loop(0, n, step1, 0)
        bstar, n_above = suffix_scan(hist1, need)

        def step2(i, _):
            k = keys_ref[pl.ds(i * 16, 16)]
            b = lax.shift_right_arithmetic(k, 24) + 128
            u2 = lax.shift_right_arithmetic(k, 16) & jnp.int32(0xFF)
            plsc.addupdate_scatter(hist2, [lane * 256 + u2], ones,
                                   mask=(b == bstar))
            return 0
        lax.fori_loop(0, n, step2, 0)
        ustar, _ = suffix_scan(hist2, need - n_above)
        return (lax.shift_left(bstar - 128, 24) | lax.shift_left(ustar, 16))

    def collect(keys_ref, read_val, read_idx, thr, n, dst_val, dst_idx):
        negv = lane.astype(jnp.float32) * 0.0 + NEG
        for q in range(CAP // 16):
            dst_val[pl.ds(q * 16, 16)] = negv
            dst_idx[pl.ds(q * 16, 16)] = lane * 0

        def step(i, off):
            k = keys_ref[pl.ds(i * 16, 16)]
            m = k >= thr
            pos = off + plsc.cumsum(m.astype(jnp.int32)) - 1
            m2 = jnp.logical_and(m, pos < CAP)
            plsc.store_scatter(dst_val, [pos], read_val(i), mask=m2)
            plsc.store_scatter(dst_idx, [pos], read_idx(i), mask=m2)
            return off + plsc.all_reduce_population_count(m)
        lax.fori_loop(0, n, step, lane * 0)

    base = wid * PER
    pltpu.sync_copy(scores_hbm.at[pl.ds(base, PER)], chunk_v)
    zero_hists()

    rv = lambda i: chunk_v[pl.ds(i * 16, 16)]
    thr = select_threshold(rv, keys_v, NV, jnp.int32(TOPK))
    collect(keys_v, rv, lambda i: base + i * 16 + lane,
            thr, NV, cand_val, cand_idx)

    pltpu.sync_copy(cand_val, shared_val.at[pl.ds(wid * CAP, CAP)])
    pltpu.sync_copy(cand_idx, shared_idx.at[pl.ds(wid * CAP, CAP)])
    plsc.subcore_barrier()

    @pl.when(wid == 0)
    def _():
        pltpu.sync_copy(shared_val, merge_val)
        pltpu.sync_copy(shared_idx, merge_idx)
        zero_hists()

        mrv = lambda i: merge_val[pl.ds(i * 16, 16)]
        mri = lambda i: merge_idx[pl.ds(i * 16, 16)]
        thr2 = select_threshold(mrv, merge_keys, MCAP // 16,
                                jnp.int32(TOPK))
        collect(merge_keys, mrv, mri, thr2, MCAP // 16,
                mcand_val, mcand_idx)

        vs = [mcand_val[pl.ds(q * 16, 16)] for q in range(CAP // 16)]
        ids = [mcand_idx[pl.ds(q * 16, 16)] for q in range(CAP // 16)]

        def extract(k, vs):
            mv = vs[0]
            for q in range(1, len(vs)):
                mv = jnp.maximum(mv, vs[q])
            m = lax.reduce_max(mv, (0,))
            imin = jnp.int32(BIG) * ones
            for q in range(len(vs)):
                imin = jnp.minimum(imin,
                                   jnp.where(vs[q] == m, ids[q], BIG))
            isel = lax.reduce_min(imin, (0,))
            plsc.store_scatter(idx32, [lane * 0 + k], lane * 0 + isel,
                               mask=(lane == 0))
            return tuple(
                jnp.where(jnp.logical_and(vs[q] == m, ids[q] == isel),
                          jnp.float32(NEG), vs[q])
                for q in range(len(vs)))
        lax.fori_loop(0, TOPK, extract, tuple(vs))

        pltpu.async_copy(mem_hbm.at[idx32], rows_v, sem).wait()
        pltpu.sync_copy(rows_v, out_hbm)


def _sc_select_gather(scores_flat, memory):
    mesh = plsc.VectorSubcoreMesh(
        core_axis_name="c", subcore_axis_name="s", num_cores=1)
    f32, i32 = jnp.float32, jnp.int32
    run = pl.kernel(
        _sc_body, mesh=mesh,
        compiler_params=pltpu.CompilerParams(needs_layout_passes=False),
        out_type=jax.ShapeDtypeStruct((TOPK, H), f32),
        scratch_types=[
            pltpu.VMEM((PER,), f32),
            pltpu.VMEM((PER,), i32),
            pltpu.VMEM((4096,), i32),
            pltpu.VMEM((4096,), i32),
            pltpu.VMEM((CAP,), f32),
            pltpu.VMEM((CAP,), i32),
            pltpu.VMEM_SHARED((MCAP,), f32),
            pltpu.VMEM_SHARED((MCAP,), i32),
            pltpu.VMEM((MCAP,), f32),
            pltpu.VMEM((MCAP,), i32),
            pltpu.VMEM((MCAP,), i32),
            pltpu.VMEM((CAP,), f32),
            pltpu.VMEM((CAP,), i32),
            pltpu.VMEM((TOPK,), i32),
            pltpu.VMEM((TOPK, H), f32),
            pltpu.SemaphoreType.DMA,
        ])
    return run(scores_flat, memory)


def kernel(x, memory):
    scores = _scores(x, memory)
    return _sc_select_gather(scores, memory)


# speedup vs baseline: 1.0010x; 1.0010x over previous
import jax
import jax.numpy as jnp
from jax import lax
from jax.experimental import pallas as pl
from jax.experimental.pallas import tpu as pltpu
from jax.experimental.pallas import tpu_sc as plsc

N_MEM = 100000
H = 128
TOPK = 32
BLK_ROWS = 8192
N_PAD = 100352
GRID = (N_MEM + BLK_ROWS - 1) // BLK_ROWS
NEG = -1e30


def _score_body(x_ref, mem_ref, out_ref, t_ref):
    pid = pl.program_id(0)

    @pl.when(pid == 0)
    def _():
        t_ref[...] = jnp.mean(x_ref[...], axis=0, keepdims=True)

    t = t_ref[...]
    m = mem_ref[...]
    num = jnp.sum(m * t, axis=1)
    ss = jnp.sum(m * m, axis=1)
    s = num * jax.lax.rsqrt(jnp.maximum(ss, jnp.float32(1e-16)))
    flat = pid * BLK_ROWS + jax.lax.iota(jnp.int32, BLK_ROWS)
    out_ref[...] = jnp.where(flat < N_MEM, s, NEG)


def _scores(x, memory):
    return pl.pallas_call(
        _score_body,
        grid=(GRID,),
        in_specs=[
            pl.BlockSpec((1024, H), lambda i: (0, 0)),
            pl.BlockSpec((BLK_ROWS, H), lambda i: (i, 0)),
        ],
        out_specs=pl.BlockSpec((BLK_ROWS,), lambda i: (i,)),
        out_shape=jax.ShapeDtypeStruct((N_PAD,), jnp.float32),
        scratch_shapes=[pltpu.VMEM((1, H), jnp.float32)],
    )(x, memory)


NT = 16
PER = N_PAD // NT
NV = PER // 16
CAP = 64
MCAP = 1024
BIG = 2**31 - 1


def _sc_body(scores_hbm, mem_hbm, out_hbm,
             chunk_v, keys_v, hist1, hist2, cand_val, cand_idx,
             shared_val, shared_idx, merge_val, merge_idx, merge_keys,
             mcand_val, mcand_idx, idx32, rows_v, sem):
    wid = lax.axis_index("s")
    lane = lax.iota(jnp.int32, 16)
    ones = lane * 0 + 1

    def keyify(v):
        u = lax.bitcast_convert_type(v, jnp.int32)
        return u ^ (lax.shift_right_arithmetic(u, 31) & jnp.int32(0x7FFFFFFF))

    def zero_hists():
        z = lane * 0
        for c in range(256):
            hist1[pl.ds(c * 16, 16)] = z
            hist2[pl.ds(c * 16, 16)] = z

    def suffix_scan(hist, need):
        def chunk_step(c2, carry):
            s_hi, found, bstar, n_above = carry
            c = 15 - c2
            acc = lane * 0
            for l in range(16):
                acc = acc + hist[pl.ds(l * 256 + c * 16, 16)]
            cs = plsc.cumsum(lax.rev(acc, dimensions=(0,)))
            tot = lax.reduce_max(cs, (0,))
            mask = (s_hi + cs) >= need
            anyv = lax.reduce_max(mask.astype(jnp.int32), (0,))
            j = lax.reduce_max(plsc.all_reduce_ffs(mask), (0,))
            below = jnp.where(lane < j, cs, lane * 0)
            n_ab = s_hi + lax.reduce_max(below, (0,))
            upd = (1 - found) * anyv
            bstar = jnp.where(upd == 1, c * 16 + 15 - j, bstar)
            n_above = jnp.where(upd == 1, n_ab, n_above)
            found = jnp.maximum(found, anyv)
            return (s_hi + tot, found, bstar, n_above)

        init = (jnp.int32(0), jnp.int32(0), jnp.int32(0), jnp.int32(0))
        _, _, bstar, n_above = lax.fori_loop(0, 16, chunk_step, init)
        return bstar, n_above

    def select_threshold(read_val, keys_ref, n, need, unroll):
        def step1(i, _):
            for u in range(unroll):
                j = i * unroll + u
                k = keyify(read_val(j))
                keys_ref[pl.ds(j * 16, 16)] = k
                b = lax.shift_right_arithmetic(k, 24) + 128
                plsc.addupdate_scatter(hist1, [lane * 256 + b], ones)
            return 0
        lax.fori_loop(0, n // unroll, step1, 0)
        bstar, n_above = suffix_scan(hist1, need)

        def step2(i, _):
            for u in range(unroll):
                j = i * unroll + u
                k = keys_ref[pl.ds(j * 16, 16)]
                b = lax.shift_right_arithmetic(k, 24) + 128
                u2 = lax.shift_right_arithmetic(k, 16) & jnp.int32(0xFF)
                plsc.addupdate_scatter(hist2, [lane * 256 + u2], ones,
                                       mask=(b == bstar))
            return 0
        lax.fori_loop(0, n // unroll, step2, 0)
        ustar, _ = suffix_scan(hist2, need - n_above)
        return (lax.shift_left(bstar - 128, 24) | lax.shift_left(ustar, 16))

    def collect(keys_ref, read_val, read_idx, thr, n, dst_val, dst_idx):
        negv = lane.astype(jnp.float32) * 0.0 + NEG
        for q in range(CAP // 16):
            dst_val[pl.ds(q * 16, 16)] = negv
            dst_idx[pl.ds(q * 16, 16)] = lane * 0

        def step(i, off):
            for u in range(2):
                j = i * 2 + u
                k = keys_ref[pl.ds(j * 16, 16)]
                m = k >= thr
                pos = off + plsc.cumsum(m.astype(jnp.int32)) - 1
                m2 = jnp.logical_and(m, pos < CAP)
                plsc.store_scatter(dst_val, [pos], read_val(j), mask=m2)
                plsc.store_scatter(dst_idx, [pos], read_idx(j), mask=m2)
                off = off + plsc.all_reduce_population_count(m)
            return off
        lax.fori_loop(0, n // 2, step, lane * 0)

    base = wid * PER
    pltpu.sync_copy(scores_hbm.at[pl.ds(base, PER)], chunk_v)
    zero_hists()

    rv = lambda i: chunk_v[pl.ds(i * 16, 16)]
    thr = select_threshold(rv, keys_v, NV, jnp.int32(TOPK), 4)
    collect(keys_v, rv, lambda i: base + i * 16 + lane,
            thr, NV, cand_val, cand_idx)

    pltpu.sync_copy(cand_val, shared_val.at[pl.ds(wid * CAP, CAP)])
    pltpu.sync_copy(cand_idx, shared_idx.at[pl.ds(wid * CAP, CAP)])
    plsc.subcore_barrier()

    @pl.when(wid == 0)
    def _():
        pltpu.sync_copy(shared_val, merge_val)
        pltpu.sync_copy(shared_idx, merge_idx)
        zero_hists()

        mrv = lambda i: merge_val[pl.ds(i * 16, 16)]
        mri = lambda i: merge_idx[pl.ds(i * 16, 16)]
        thr2 = select_threshold(mrv, merge_keys, MCAP // 16,
                                jnp.int32(TOPK), 4)
        collect(merge_keys, mrv, mri, thr2, MCAP // 16,
                mcand_val, mcand_idx)

        vs = [mcand_val[pl.ds(q * 16, 16)] for q in range(CAP // 16)]
        ids = [mcand_idx[pl.ds(q * 16, 16)] for q in range(CAP // 16)]

        def extract(k, vs):
            mv = vs[0]
            for q in range(1, len(vs)):
                mv = jnp.maximum(mv, vs[q])
            m = lax.reduce_max(mv, (0,))
            imin = jnp.int32(BIG) * ones
            for q in range(len(vs)):
                imin = jnp.minimum(imin,
                                   jnp.where(vs[q] == m, ids[q], BIG))
            isel = lax.reduce_min(imin, (0,))
            plsc.store_scatter(idx32, [lane * 0 + k], lane * 0 + isel,
                               mask=(lane == 0))
            return tuple(
                jnp.where(jnp.logical_and(vs[q] == m, ids[q] == isel),
                          jnp.float32(NEG), vs[q])
                for q in range(len(vs)))
        lax.fori_loop(0, TOPK, extract, tuple(vs))

        pltpu.async_copy(mem_hbm.at[idx32], rows_v, sem).wait()
        pltpu.sync_copy(rows_v, out_hbm)


def _sc_select_gather(scores_flat, memory):
    mesh = plsc.VectorSubcoreMesh(
        core_axis_name="c", subcore_axis_name="s", num_cores=1)
    f32, i32 = jnp.float32, jnp.int32
    run = pl.kernel(
        _sc_body, mesh=mesh,
        compiler_params=pltpu.CompilerParams(needs_layout_passes=False),
        out_type=jax.ShapeDtypeStruct((TOPK, H), f32),
        scratch_types=[
            pltpu.VMEM((PER,), f32),
            pltpu.VMEM((PER,), i32),
            pltpu.VMEM((4096,), i32),
            pltpu.VMEM((4096,), i32),
            pltpu.VMEM((CAP,), f32),
            pltpu.VMEM((CAP,), i32),
            pltpu.VMEM_SHARED((MCAP,), f32),
            pltpu.VMEM_SHARED((MCAP,), i32),
            pltpu.VMEM((MCAP,), f32),
            pltpu.VMEM((MCAP,), i32),
            pltpu.VMEM((MCAP,), i32),
            pltpu.VMEM((CAP,), f32),
            pltpu.VMEM((CAP,), i32),
            pltpu.VMEM((TOPK,), i32),
            pltpu.VMEM((TOPK, H), f32),
            pltpu.SemaphoreType.DMA,
        ])
    return run(scores_flat, memory)


def kernel(x, memory):
    scores = _scores(x, memory)
    return _sc_select_gather(scores, memory)


# speedup vs baseline: 1.0151x; 1.0141x over previous
import jax
import jax.numpy as jnp
from jax import lax
from jax.experimental import pallas as pl
from jax.experimental.pallas import tpu as pltpu
from jax.experimental.pallas import tpu_sc as plsc

N_MEM = 100000
H = 128
TOPK = 32
BLK_ROWS = 8192
N_PAD = 100352
GRID = (N_MEM + BLK_ROWS - 1) // BLK_ROWS
NEG = -1e30


def _score_body(x_ref, mem_ref, out_ref, t_ref):
    pid = pl.program_id(0)

    @pl.when(pid == 0)
    def _():
        t_ref[...] = jnp.mean(x_ref[...], axis=0, keepdims=True)

    t = t_ref[...]
    m = mem_ref[...]
    num = jnp.sum(m * t, axis=1)
    ss = jnp.sum(m * m, axis=1)
    s = num * jax.lax.rsqrt(jnp.maximum(ss, jnp.float32(1e-16)))
    flat = pid * BLK_ROWS + jax.lax.iota(jnp.int32, BLK_ROWS)
    out_ref[...] = jnp.where(flat < N_MEM, s, NEG)


def _scores(x, memory):
    return pl.pallas_call(
        _score_body,
        grid=(GRID,),
        in_specs=[
            pl.BlockSpec((1024, H), lambda i: (0, 0)),
            pl.BlockSpec((BLK_ROWS, H), lambda i: (i, 0)),
        ],
        out_specs=pl.BlockSpec((BLK_ROWS,), lambda i: (i,)),
        out_shape=jax.ShapeDtypeStruct((N_PAD,), jnp.float32),
        scratch_shapes=[pltpu.VMEM((1, H), jnp.float32)],
    )(x, memory)


NT = 16
PER = N_PAD // NT
NV = PER // 16
CAP = 64
MCAP = 1024
BIG = 2**31 - 1


def _sc_body(scores_hbm, mem_hbm, out_hbm,
             chunk_v, keys_v, hist1, hist2, cand_val, cand_idx,
             shared_val, shared_idx, merge_val, merge_idx, merge_keys,
             mcand_val, mcand_idx, idx32, rows_v, sem):
    wid = lax.axis_index("s")
    lane = lax.iota(jnp.int32, 16)
    ones = lane * 0 + 1

    def keyify(v):
        u = lax.bitcast_convert_type(v, jnp.int32)
        return u ^ (lax.shift_right_arithmetic(u, 31) & jnp.int32(0x7FFFFFFF))

    def zero_hists():
        z = lane * 0
        for c in range(256):
            hist1[pl.ds(c * 16, 16)] = z
            hist2[pl.ds(c * 16, 16)] = z

    def suffix_scan(hist, need):
        def chunk_tot(c, ct):
            acc = lane * 0
            for l in range(16):
                acc = acc + hist[pl.ds(c * 256 + l * 16, 16)]
            tot = lax.reduce_sum(acc, (0,))
            return jnp.where(lane == c, tot, ct)
        ct = lax.fori_loop(0, 16, chunk_tot, lane * 0)
        csc = plsc.cumsum(lax.rev(ct, dimensions=(0,)))
        jc = lax.reduce_max(plsc.all_reduce_ffs(csc >= need), (0,))
        cstar = 15 - jc
        s_hi = lax.reduce_max(jnp.where(lane < jc, csc, lane * 0), (0,))

        def bucket_tot(l, bt):
            v = hist[pl.ds(cstar * 256 + l * 16, 16)]
            return jnp.where(lane == l, lax.reduce_sum(v, (0,)), bt)
        bt = lax.fori_loop(0, 16, bucket_tot, lane * 0)
        cs = plsc.cumsum(lax.rev(bt, dimensions=(0,)))
        j = lax.reduce_max(plsc.all_reduce_ffs((s_hi + cs) >= need), (0,))
        bstar = cstar * 16 + 15 - j
        n_above = s_hi + lax.reduce_max(
            jnp.where(lane < j, cs, lane * 0), (0,))
        return bstar, n_above

    def select_threshold(read_val, keys_ref, n, need, unroll):
        def step1(i, _):
            for u in range(unroll):
                j = i * unroll + u
                k = keyify(read_val(j))
                keys_ref[pl.ds(j * 16, 16)] = k
                b = lax.shift_right_arithmetic(k, 24) + 128
                plsc.addupdate_scatter(hist1, [b * 16 + lane], ones)
            return 0
        lax.fori_loop(0, n // unroll, step1, 0)
        bstar, n_above = suffix_scan(hist1, need)

        def step2(i, _):
            for u in range(unroll):
                j = i * unroll + u
                k = keys_ref[pl.ds(j * 16, 16)]
                b = lax.shift_right_arithmetic(k, 24) + 128
                u2 = lax.shift_right_arithmetic(k, 16) & jnp.int32(0xFF)
                plsc.addupdate_scatter(hist2, [u2 * 16 + lane], ones,
                                       mask=(b == bstar))
            return 0
        lax.fori_loop(0, n // unroll, step2, 0)
        ustar, _ = suffix_scan(hist2, need - n_above)
        return (lax.shift_left(bstar - 128, 24) | lax.shift_left(ustar, 16))

    def collect(keys_ref, read_val, read_idx, thr, n, dst_val, dst_idx):
        negv = lane.astype(jnp.float32) * 0.0 + NEG
        for q in range(CAP // 16):
            dst_val[pl.ds(q * 16, 16)] = negv
            dst_idx[pl.ds(q * 16, 16)] = lane * 0

        def step(i, off):
            for u in range(2):
                j = i * 2 + u
                k = keys_ref[pl.ds(j * 16, 16)]
                m = k >= thr
                pos = off + plsc.cumsum(m.astype(jnp.int32)) - 1
                m2 = jnp.logical_and(m, pos < CAP)
                plsc.store_scatter(dst_val, [pos], read_val(j), mask=m2)
                plsc.store_scatter(dst_idx, [pos], read_idx(j), mask=m2)
                off = off + plsc.all_reduce_population_count(m)
            return off
        lax.fori_loop(0, n // 2, step, lane * 0)

    base = wid * PER
    pltpu.sync_copy(scores_hbm.at[pl.ds(base, PER)], chunk_v)
    zero_hists()

    rv = lambda i: chunk_v[pl.ds(i * 16, 16)]
    thr = select_threshold(rv, keys_v, NV, jnp.int32(TOPK), 4)
    collect(keys_v, rv, lambda i: base + i * 16 + lane,
            thr, NV, cand_val, cand_idx)

    pltpu.sync_copy(cand_val, shared_val.at[pl.ds(wid * CAP, CAP)])
    pltpu.sync_copy(cand_idx, shared_idx.at[pl.ds(wid * CAP, CAP)])
    plsc.subcore_barrier()

    @pl.when(wid == 0)
    def _():
        pltpu.sync_copy(shared_val, merge_val)
        pltpu.sync_copy(shared_idx, merge_idx)
        zero_hists()

        mrv = lambda i: merge_val[pl.ds(i * 16, 16)]
        mri = lambda i: merge_idx[pl.ds(i * 16, 16)]
        thr2 = select_threshold(mrv, merge_keys, MCAP // 16,
                                jnp.int32(TOPK), 4)
        collect(merge_keys, mrv, mri, thr2, MCAP // 16,
                mcand_val, mcand_idx)

        vs = [mcand_val[pl.ds(q * 16, 16)] for q in range(CAP // 16)]
        ids = [mcand_idx[pl.ds(q * 16, 16)] for q in range(CAP // 16)]

        def extract(k, vs):
            mv = vs[0]
            for q in range(1, len(vs)):
                mv = jnp.maximum(mv, vs[q])
            m = lax.reduce_max(mv, (0,))
            imin = jnp.int32(BIG) * ones
            for q in range(len(vs)):
                imin = jnp.minimum(imin,
                                   jnp.where(vs[q] == m, ids[q], BIG))
            isel = lax.reduce_min(imin, (0,))
            plsc.store_scatter(idx32, [lane * 0 + k], lane * 0 + isel,
                               mask=(lane == 0))
            return tuple(
                jnp.where(jnp.logical_and(vs[q] == m, ids[q] == isel),
                          jnp.float32(NEG), vs[q])
                for q in range(len(vs)))
        lax.fori_loop(0, TOPK, extract, tuple(vs))

        pltpu.async_copy(mem_hbm.at[idx32], rows_v, sem).wait()
        pltpu.sync_copy(rows_v, out_hbm)


def _sc_select_gather(scores_flat, memory):
    mesh = plsc.VectorSubcoreMesh(
        core_axis_name="c", subcore_axis_name="s", num_cores=1)
    f32, i32 = jnp.float32, jnp.int32
    run = pl.kernel(
        _sc_body, mesh=mesh,
        compiler_params=pltpu.CompilerParams(needs_layout_passes=False),
        out_type=jax.ShapeDtypeStruct((TOPK, H), f32),
        scratch_types=[
            pltpu.VMEM((PER,), f32),
            pltpu.VMEM((PER,), i32),
            pltpu.VMEM((4096,), i32),
            pltpu.VMEM((4096,), i32),
            pltpu.VMEM((CAP,), f32),
            pltpu.VMEM((CAP,), i32),
            pltpu.VMEM_SHARED((MCAP,), f32),
            pltpu.VMEM_SHARED((MCAP,), i32),
            pltpu.VMEM((MCAP,), f32),
            pltpu.VMEM((MCAP,), i32),
            pltpu.VMEM((MCAP,), i32),
            pltpu.VMEM((CAP,), f32),
            pltpu.VMEM((CAP,), i32),
            pltpu.VMEM((TOPK,), i32),
            pltpu.VMEM((TOPK, H), f32),
            pltpu.SemaphoreType.DMA,
        ])
    return run(scores_flat, memory)


def kernel(x, memory):
    scores = _scores(x, memory)
    return _sc_select_gather(scores, memory)
